# baseline (speedup 1.0000x reference)
import functools

import jax
import jax.numpy as jnp
from jax.experimental import pallas as pl
from jax.experimental.pallas import tpu as pltpu


def _fused_body(nbr1, nbr2, node,
                w_self1, b_self1, w_nbr1, b_nbr1, g1, be1,
                w_self2, b_self2, w_nbr2, b_nbr2, g2, be2,
                out,
                acc1, acc2, c1s, c2s, ssq, buf,
                *, R, D, n_nbr, dense_blk, eps):
    i = pl.program_id(0)

    @pl.when(i == 0)
    def _init():
        acc1[...] = jnp.zeros_like(acc1)
        acc2[...] = jnp.zeros_like(acc2)
        ssq[...] = jnp.zeros_like(ssq)

    @pl.when(i < R)
    def _reduce():
        acc1[...] += jnp.sum(nbr1[...], axis=0, keepdims=True)
        acc2[...] += jnp.sum(nbr2[...], axis=0, keepdims=True)

    @pl.when(i < D)
    def _prehoist():
        buf[pl.ds(i * dense_blk, dense_blk), :] = jnp.dot(
            node[...], w_self1[...], preferred_element_type=jnp.float32)

    @pl.when(i == R)
    def _bias():
        agg1 = acc1[...] * (1.0 / n_nbr)
        agg2 = acc2[...] * (1.0 / n_nbr)
        c1s[...] = (jnp.dot(agg1, w_nbr1[...], preferred_element_type=jnp.float32)
                    + b_self1[...] + b_nbr1[...])
        c2s[...] = (jnp.dot(agg2, w_nbr2[...], preferred_element_type=jnp.float32)
                    + b_self2[...] + b_nbr2[...])

    @pl.when((i >= R) & (i < R + D))
    def _dense():
        j = i - R
        x = buf[pl.ds(j * dense_blk, dense_blk), :] + c1s[...]
        mu = jnp.mean(x, axis=-1, keepdims=True)
        var = jnp.mean((x - mu) ** 2, axis=-1, keepdims=True)
        x = (x - mu) * jax.lax.rsqrt(var + eps) * g1[...] + be1[...]
        h1 = jnp.maximum(x, 0.0)
        y = jnp.dot(h1, w_self2[...], preferred_element_type=jnp.float32)
        y = y + c2s[...]
        mu2 = jnp.mean(y, axis=-1, keepdims=True)
        var2 = jnp.mean((y - mu2) ** 2, axis=-1, keepdims=True)
        y = (y - mu2) * jax.lax.rsqrt(var2 + eps) * g2[...] + be2[...]
        h2 = jnp.maximum(y, 0.0)
        buf[pl.ds(j * dense_blk, dense_blk), :] = h2
        ssq[...] += jnp.sum(h2 * h2, axis=0, keepdims=True)

    @pl.when(i >= R + D)
    def _emit():
        j = i - (R + D)
        inv = 1.0 / jnp.maximum(jnp.sqrt(ssq[...]), 1e-12)
        out[...] = buf[pl.ds(j * dense_blk, dense_blk), :] * inv


def kernel(node_feat, neighbor_feats_l1, neighbor_feats_l2,
           W_self1, b_self1, W_nbr1, b_nbr1, g1, be1,
           W_self2, b_self2, W_nbr2, b_nbr2, g2, be2):
    n_nodes, feat = node_feat.shape
    n_nbr = neighbor_feats_l1.shape[0]
    hid = W_self1.shape[1]
    emb = W_self2.shape[1]

    nbr_blk = 10000 if n_nbr % 10000 == 0 else n_nbr
    dense_blk = 1000 if n_nodes % 1000 == 0 else n_nodes
    R = n_nbr // nbr_blk
    D = n_nodes // dense_blk
    grid = (R + 2 * D,)

    b_self1 = b_self1.reshape(1, hid)
    b_nbr1 = b_nbr1.reshape(1, hid)
    g1 = g1.reshape(1, hid)
    be1 = be1.reshape(1, hid)
    b_self2 = b_self2.reshape(1, emb)
    b_nbr2 = b_nbr2.reshape(1, emb)
    g2 = g2.reshape(1, emb)
    be2 = be2.reshape(1, emb)

    def nbr_map(i):
        return (jnp.minimum(i, R - 1), 0)

    def node_map(i):
        return (jnp.minimum(i, D - 1), 0)

    def out_map(i):
        return (jnp.clip(i - (R + D), 0, D - 1), 0)

    full = lambda s: pl.BlockSpec(s, lambda i: (0, 0))

    body = functools.partial(_fused_body, R=R, D=D, n_nbr=n_nbr,
                             dense_blk=dense_blk, eps=1e-5)

    return pl.pallas_call(
        body,
        grid=grid,
        in_specs=[
            pl.BlockSpec((nbr_blk, feat), nbr_map),
            pl.BlockSpec((nbr_blk, hid), nbr_map),
            pl.BlockSpec((dense_blk, feat), node_map),
            full((feat, hid)), full((1, hid)), full((feat, hid)), full((1, hid)),
            full((1, hid)), full((1, hid)),
            full((hid, emb)), full((1, emb)), full((hid, emb)), full((1, emb)),
            full((1, emb)), full((1, emb)),
        ],
        out_specs=pl.BlockSpec((dense_blk, emb), out_map),
        out_shape=jax.ShapeDtypeStruct((n_nodes, emb), jnp.float32),
        scratch_shapes=[
            pltpu.VMEM((1, feat), jnp.float32),
            pltpu.VMEM((1, hid), jnp.float32),
            pltpu.VMEM((1, hid), jnp.float32),
            pltpu.VMEM((1, emb), jnp.float32),
            pltpu.VMEM((1, emb), jnp.float32),
            pltpu.VMEM((n_nodes, emb), jnp.float32),
        ],
        compiler_params=pltpu.CompilerParams(
            dimension_semantics=("arbitrary",),
        ),
    )(neighbor_feats_l1, neighbor_feats_l2, node_feat,
      W_self1, b_self1, W_nbr1, b_nbr1, g1, be1,
      W_self2, b_self2, W_nbr2, b_nbr2, g2, be2)


# speedup vs baseline: 1.0188x; 1.0188x over previous
import functools

import jax
import jax.numpy as jnp
from jax.experimental import pallas as pl
from jax.experimental.pallas import tpu as pltpu


def _fused_body(nbr1, nbr2, node,
                w_self1, b_self1, w_nbr1, b_nbr1, g1, be1,
                w_self2, b_self2, w_nbr2, b_nbr2, g2, be2,
                out,
                acc1, acc2, c1s, c2s, ssq, buf,
                *, R, D, n_nbr, dense_blk, eps):
    i = pl.program_id(0)

    @pl.when(i == 0)
    def _init():
        acc1[...] = jnp.zeros_like(acc1)
        acc2[...] = jnp.zeros_like(acc2)
        ssq[...] = jnp.zeros_like(ssq)

    @pl.when(i < R)
    def _reduce():
        acc1[...] += jnp.sum(nbr1[...], axis=0, keepdims=True)
        acc2[...] += jnp.sum(nbr2[...], axis=0, keepdims=True)

    @pl.when(i < D)
    def _prehoist():
        buf[pl.ds(i * dense_blk, dense_blk), :] = jnp.dot(
            node[...], w_self1[...], preferred_element_type=jnp.float32)

    @pl.when(i == R)
    def _bias():
        agg1 = acc1[...] * (1.0 / n_nbr)
        agg2 = acc2[...] * (1.0 / n_nbr)
        c1s[...] = (jnp.dot(agg1, w_nbr1[...], preferred_element_type=jnp.float32)
                    + b_self1[...] + b_nbr1[...])
        c2s[...] = (jnp.dot(agg2, w_nbr2[...], preferred_element_type=jnp.float32)
                    + b_self2[...] + b_nbr2[...])

    @pl.when((i >= R) & (i < R + D))
    def _dense():
        j = i - R
        x = buf[pl.ds(j * dense_blk, dense_blk), :] + c1s[...]
        mu = jnp.mean(x, axis=-1, keepdims=True)
        var = jnp.mean((x - mu) ** 2, axis=-1, keepdims=True)
        x = (x - mu) * jax.lax.rsqrt(var + eps) * g1[...] + be1[...]
        h1 = jnp.maximum(x, 0.0)
        y = jnp.dot(h1, w_self2[...], preferred_element_type=jnp.float32)
        y = y + c2s[...]
        mu2 = jnp.mean(y, axis=-1, keepdims=True)
        var2 = jnp.mean((y - mu2) ** 2, axis=-1, keepdims=True)
        y = (y - mu2) * jax.lax.rsqrt(var2 + eps) * g2[...] + be2[...]
        h2 = jnp.maximum(y, 0.0)
        buf[pl.ds(j * dense_blk, dense_blk), :] = h2
        ssq[...] += jnp.sum(h2 * h2, axis=0, keepdims=True)

    @pl.when(i >= R + D)
    def _emit():
        j = i - (R + D)
        inv = 1.0 / jnp.maximum(jnp.sqrt(ssq[...]), 1e-12)
        out[...] = buf[pl.ds(j * dense_blk, dense_blk), :] * inv


def kernel(node_feat, neighbor_feats_l1, neighbor_feats_l2,
           W_self1, b_self1, W_nbr1, b_nbr1, g1, be1,
           W_self2, b_self2, W_nbr2, b_nbr2, g2, be2):
    n_nodes, feat = node_feat.shape
    n_nbr = neighbor_feats_l1.shape[0]
    hid = W_self1.shape[1]
    emb = W_self2.shape[1]

    nbr_blk = 8000 if n_nbr % 8000 == 0 else n_nbr
    dense_blk = 2000 if n_nodes % 2000 == 0 else n_nodes
    R = n_nbr // nbr_blk
    D = n_nodes // dense_blk
    grid = (R + 2 * D,)

    b_self1 = b_self1.reshape(1, hid)
    b_nbr1 = b_nbr1.reshape(1, hid)
    g1 = g1.reshape(1, hid)
    be1 = be1.reshape(1, hid)
    b_self2 = b_self2.reshape(1, emb)
    b_nbr2 = b_nbr2.reshape(1, emb)
    g2 = g2.reshape(1, emb)
    be2 = be2.reshape(1, emb)

    def nbr_map(i):
        return (jnp.minimum(i, R - 1), 0)

    def node_map(i):
        return (jnp.minimum(i, D - 1), 0)

    def out_map(i):
        return (jnp.clip(i - (R + D), 0, D - 1), 0)

    full = lambda s: pl.BlockSpec(s, lambda i: (0, 0))

    body = functools.partial(_fused_body, R=R, D=D, n_nbr=n_nbr,
                             dense_blk=dense_blk, eps=1e-5)

    return pl.pallas_call(
        body,
        grid=grid,
        in_specs=[
            pl.BlockSpec((nbr_blk, feat), nbr_map),
            pl.BlockSpec((nbr_blk, hid), nbr_map),
            pl.BlockSpec((dense_blk, feat), node_map),
            full((feat, hid)), full((1, hid)), full((feat, hid)), full((1, hid)),
            full((1, hid)), full((1, hid)),
            full((hid, emb)), full((1, emb)), full((hid, emb)), full((1, emb)),
            full((1, emb)), full((1, emb)),
        ],
        out_specs=pl.BlockSpec((dense_blk, emb), out_map),
        out_shape=jax.ShapeDtypeStruct((n_nodes, emb), jnp.float32),
        scratch_shapes=[
            pltpu.VMEM((1, feat), jnp.float32),
            pltpu.VMEM((1, hid), jnp.float32),
            pltpu.VMEM((1, hid), jnp.float32),
            pltpu.VMEM((1, emb), jnp.float32),
            pltpu.VMEM((1, emb), jnp.float32),
            pltpu.VMEM((n_nodes, emb), jnp.float32),
        ],
        compiler_params=pltpu.CompilerParams(
            dimension_semantics=("arbitrary",),
        ),
    )(neighbor_feats_l1, neighbor_feats_l2, node_feat,
      W_self1, b_self1, W_nbr1, b_nbr1, g1, be1,
      W_self2, b_self2, W_nbr2, b_nbr2, g2, be2)


# speedup vs baseline: 1.0251x; 1.0061x over previous
import functools

import jax
import jax.numpy as jnp
from jax.experimental import pallas as pl
from jax.experimental.pallas import tpu as pltpu


def _fused_body(nbr1, nbr2, node,
                w_self1, b_self1, w_nbr1, b_nbr1, g1, be1,
                w_self2, b_self2, w_nbr2, b_nbr2, g2, be2,
                out,
                acc1, acc2, c1s, c2s, ssq, buf,
                *, R, D, n_nbr, dense_blk, eps):
    i = pl.program_id(0)

    @pl.when(i == 0)
    def _init():
        acc1[...] = jnp.zeros_like(acc1)
        acc2[...] = jnp.zeros_like(acc2)
        ssq[...] = jnp.zeros_like(ssq)

    @pl.when(i < R)
    def _reduce():
        acc1[...] += jnp.sum(nbr1[...], axis=0, keepdims=True)
        acc2[...] += jnp.sum(nbr2[...], axis=0, keepdims=True)

    @pl.when(i < D)
    def _prehoist():
        buf[pl.ds(i * dense_blk, dense_blk), :] = jnp.dot(
            node[...], w_self1[...], preferred_element_type=jnp.float32)

    @pl.when(i == R)
    def _bias():
        agg1 = acc1[...] * (1.0 / n_nbr)
        agg2 = acc2[...] * (1.0 / n_nbr)
        c1s[...] = (jnp.dot(agg1, w_nbr1[...], preferred_element_type=jnp.float32)
                    + b_self1[...] + b_nbr1[...])
        c2s[...] = (jnp.dot(agg2, w_nbr2[...], preferred_element_type=jnp.float32)
                    + b_self2[...] + b_nbr2[...])

    @pl.when((i >= R) & (i < R + D))
    def _dense():
        j = i - R
        x = buf[pl.ds(j * dense_blk, dense_blk), :] + c1s[...]
        mu = jnp.mean(x, axis=-1, keepdims=True)
        var = jnp.mean((x - mu) ** 2, axis=-1, keepdims=True)
        x = (x - mu) * jax.lax.rsqrt(var + eps) * g1[...] + be1[...]
        h1 = jnp.maximum(x, 0.0)
        y = jnp.dot(h1, w_self2[...], preferred_element_type=jnp.float32)
        y = y + c2s[...]
        mu2 = jnp.mean(y, axis=-1, keepdims=True)
        var2 = jnp.mean((y - mu2) ** 2, axis=-1, keepdims=True)
        y = (y - mu2) * jax.lax.rsqrt(var2 + eps) * g2[...] + be2[...]
        h2 = jnp.maximum(y, 0.0)
        buf[pl.ds(j * dense_blk, dense_blk), :] = h2
        ssq[...] += jnp.sum(h2 * h2, axis=0, keepdims=True)

    @pl.when(i >= R + D)
    def _emit():
        j = i - (R + D)
        inv = 1.0 / jnp.maximum(jnp.sqrt(ssq[...]), 1e-12)
        out[...] = buf[pl.ds(j * dense_blk, dense_blk), :] * inv


def kernel(node_feat, neighbor_feats_l1, neighbor_feats_l2,
           W_self1, b_self1, W_nbr1, b_nbr1, g1, be1,
           W_self2, b_self2, W_nbr2, b_nbr2, g2, be2):
    n_nodes, feat = node_feat.shape
    n_nbr = neighbor_feats_l1.shape[0]
    hid = W_self1.shape[1]
    emb = W_self2.shape[1]

    nbr_blk = 8000 if n_nbr % 8000 == 0 else n_nbr
    dense_blk = 5000 if n_nodes % 5000 == 0 else n_nodes
    R = n_nbr // nbr_blk
    D = n_nodes // dense_blk
    grid = (R + 2 * D,)

    b_self1 = b_self1.reshape(1, hid)
    b_nbr1 = b_nbr1.reshape(1, hid)
    g1 = g1.reshape(1, hid)
    be1 = be1.reshape(1, hid)
    b_self2 = b_self2.reshape(1, emb)
    b_nbr2 = b_nbr2.reshape(1, emb)
    g2 = g2.reshape(1, emb)
    be2 = be2.reshape(1, emb)

    def nbr_map(i):
        return (jnp.minimum(i, R - 1), 0)

    def node_map(i):
        return (jnp.minimum(i, D - 1), 0)

    def out_map(i):
        return (jnp.clip(i - (R + D), 0, D - 1), 0)

    full = lambda s: pl.BlockSpec(s, lambda i: (0, 0))

    body = functools.partial(_fused_body, R=R, D=D, n_nbr=n_nbr,
                             dense_blk=dense_blk, eps=1e-5)

    return pl.pallas_call(
        body,
        grid=grid,
        in_specs=[
            pl.BlockSpec((nbr_blk, feat), nbr_map),
            pl.BlockSpec((nbr_blk, hid), nbr_map),
            pl.BlockSpec((dense_blk, feat), node_map),
            full((feat, hid)), full((1, hid)), full((feat, hid)), full((1, hid)),
            full((1, hid)), full((1, hid)),
            full((hid, emb)), full((1, emb)), full((hid, emb)), full((1, emb)),
            full((1, emb)), full((1, emb)),
        ],
        out_specs=pl.BlockSpec((dense_blk, emb), out_map),
        out_shape=jax.ShapeDtypeStruct((n_nodes, emb), jnp.float32),
        scratch_shapes=[
            pltpu.VMEM((1, feat), jnp.float32),
            pltpu.VMEM((1, hid), jnp.float32),
            pltpu.VMEM((1, hid), jnp.float32),
            pltpu.VMEM((1, emb), jnp.float32),
            pltpu.VMEM((1, emb), jnp.float32),
            pltpu.VMEM((n_nodes, emb), jnp.float32),
        ],
        compiler_params=pltpu.CompilerParams(
            dimension_semantics=("arbitrary",),
        ),
    )(neighbor_feats_l1, neighbor_feats_l2, node_feat,
      W_self1, b_self1, W_nbr1, b_nbr1, g1, be1,
      W_self2, b_self2, W_nbr2, b_nbr2, g2, be2)
